# Initial kernel scaffold; baseline (speedup 1.0000x reference)
import jax
import jax.numpy as jnp
from jax.experimental import pallas as pl


def kernel(obj_vecs, pred_vecs, edges, W, b):
    raise NotImplementedError("write your pallas kernel here")


# speedup vs baseline: 36.4515x; 36.4515x over previous
import functools

import jax
import jax.numpy as jnp
from jax import lax
from jax.experimental import pallas as pl
from jax.experimental.pallas import tpu as pltpu
from jax.experimental.pallas import tpu_sc as plsc

O = 10000
T = 320000
D = 128
N = O + T

NC, NS = 2, 16
NW = NC * NS
EPW = T // NW
CH = 80
NCH = EPW // CH
OPAD = 10240
RPT = OPAD // NS
INV_SQRT2 = 0.7071067811865476
SQRT2 = 1.4142135623730951

BR = 1000
NBO = O // BR
NBP = T // BR
NBMID = (T - O) // BR

_MESH = plsc.VectorSubcoreMesh(core_axis_name="c", subcore_axis_name="s")


def _hist_body(o3, hist_out, hist_sh, idx_v, ones_v, zbuf):
    core = lax.axis_index("c")
    sub = lax.axis_index("s")
    wid = sub * NC + core

    def _z(i, carry):
        zbuf[pl.ds(i * 16, 16)] = jnp.zeros((16,), jnp.int32)
        return carry

    lax.fori_loop(0, RPT // 16, _z, 0)

    def _o(i, carry):
        ones_v[pl.ds(i * 16, 16)] = jnp.ones((16,), jnp.int32)
        return carry

    lax.fori_loop(0, CH // 16, _o, 0)

    pltpu.sync_copy(zbuf, hist_sh.at[pl.ds(sub * RPT, RPT)])
    plsc.subcore_barrier()

    pltpu.sync_copy(o3.at[wid], idx_v)

    def _step(c, carry):
        pltpu.sync_copy(ones_v, hist_sh.at[idx_v.at[c]], add=True)
        return carry

    lax.fori_loop(0, NCH, _step, 0)
    plsc.subcore_barrier()
    pltpu.sync_copy(
        hist_sh.at[pl.ds(sub * RPT, RPT)],
        hist_out.at[core, pl.ds(sub * RPT, RPT)],
    )


_hist_call = pl.kernel(
    _hist_body,
    out_type=jax.ShapeDtypeStruct((NC, OPAD), jnp.int32),
    mesh=_MESH,
    scratch_types=[
        pltpu.VMEM_SHARED((OPAD,), jnp.int32),
        pltpu.VMEM((NCH, CH), jnp.int32),
        pltpu.VMEM((CH,), jnp.int32),
        pltpu.VMEM((RPT,), jnp.int32),
    ],
)


def _prep_body(hist, obj, scaled, scaled2, dinv, invdeg):
    h = hist[...].astype(jnp.float32)
    ones = jnp.ones((NC, 1), jnp.float32)
    deg = lax.dot_general(h, ones, (((0,), (0,)), ((), ())),
                          preferred_element_type=jnp.float32) + 2.0
    dv = lax.rsqrt(deg)
    dinv[...] = dv
    invdeg[...] = 1.0 / deg
    s = obj[...] * dv
    scaled[...] = s
    scaled2[...] = s * SQRT2


BRK = 2048

_prep_call = pl.pallas_call(
    _prep_body,
    grid=(OPAD // BRK,),
    in_specs=[
        pl.BlockSpec((NC, BRK), lambda i: (0, i)),
        pl.BlockSpec((BRK, D), lambda i: (i, 0)),
    ],
    out_specs=[
        pl.BlockSpec((BRK, D), lambda i: (i, 0)),
        pl.BlockSpec((BRK, D), lambda i: (i, 0)),
        pl.BlockSpec((BRK, 1), lambda i: (i, 0)),
        pl.BlockSpec((BRK, 1), lambda i: (i, 0)),
    ],
    out_shape=[
        jax.ShapeDtypeStruct((O, D), jnp.float32),
        jax.ShapeDtypeStruct((O, D), jnp.float32),
        jax.ShapeDtypeStruct((O, 1), jnp.float32),
        jax.ShapeDtypeStruct((O, 1), jnp.float32),
    ],
)


def _gather_body(tab, s3, y1, idx_v, rows, sem):
    core = lax.axis_index("c")
    sub = lax.axis_index("s")
    wid = sub * NC + core
    pltpu.sync_copy(s3.at[wid], idx_v)

    def _step(c, carry):
        pltpu.async_copy(tab.at[idx_v.at[c]], rows, sem).wait()
        pltpu.sync_copy(rows, y1.at[pl.ds(wid * EPW + c * CH, CH)])
        return carry

    lax.fori_loop(0, NCH, _step, 0)


_gather_call = pl.kernel(
    _gather_body,
    out_type=jax.ShapeDtypeStruct((T, D), jnp.float32),
    mesh=_MESH,
    scratch_types=[
        pltpu.VMEM((NCH, CH), jnp.int32),
        pltpu.VMEM((CH, D), jnp.float32),
        pltpu.SemaphoreType.DMA,
    ],
)


def _scatter_body(pred, scaled2, o3, acc_out, acc, idx_v, rows, zbuf):
    core = lax.axis_index("c")
    sub = lax.axis_index("s")
    wid = sub * NC + core

    def _z(i, carry):
        r = i // 8
        cc = (i % 8) * 16
        zbuf[r, pl.ds(cc, 16)] = jnp.zeros((16,), jnp.float32)
        return carry

    lax.fori_loop(0, CH * 8, _z, 0)
    for j in range(RPT // CH):
        pltpu.sync_copy(zbuf, acc.at[pl.ds(sub * RPT + j * CH, CH)])
    plsc.subcore_barrier()

    pltpu.sync_copy(o3.at[wid], idx_v)

    def _step(c, carry):
        @pl.when(wid == 0)
        def _():
            pltpu.sync_copy(scaled2.at[pl.ds(c * CH, CH)], rows)

        @pl.when(wid != 0)
        def _():
            pltpu.sync_copy(pred.at[pl.ds(wid * EPW - O + c * CH, CH)], rows)

        pltpu.sync_copy(rows, acc.at[idx_v.at[c]], add=True)
        return carry

    lax.fori_loop(0, NCH, _step, 0)
    plsc.subcore_barrier()
    for j in range(RPT // CH):
        sl = pl.ds(sub * RPT + j * CH, CH)
        pltpu.sync_copy(acc.at[sl], acc_out.at[core, sl])


_scatter_call = pl.kernel(
    _scatter_body,
    out_type=jax.ShapeDtypeStruct((NC, OPAD, D), jnp.float32),
    mesh=_MESH,
    scratch_types=[
        pltpu.VMEM_SHARED((OPAD, D), jnp.float32),
        pltpu.VMEM((NCH, CH), jnp.int32),
        pltpu.VMEM((CH, D), jnp.float32),
        pltpu.VMEM((CH, D), jnp.float32),
    ],
)


def _obj_body(obj, y1, accs, dinv, invdeg, Wm, bm, out):
    a = accs[...]
    dv = dinv[...]
    y = (obj[...] * invdeg[...]
         + y1[...] * dv
         + (a[0] + a[1]) * (dv * INV_SQRT2))
    out[...] = lax.dot_general(
        y, Wm[...], (((1,), (1,)), ((), ())),
        preferred_element_type=jnp.float32) + bm[...]


_obj_call = pl.pallas_call(
    _obj_body,
    grid=(NBO,),
    in_specs=[
        pl.BlockSpec((BR, D), lambda i: (i, 0)),
        pl.BlockSpec((BR, D), lambda i: (i, 0)),
        pl.BlockSpec((NC, BR, D), lambda i: (0, i, 0)),
        pl.BlockSpec((BR, 1), lambda i: (i, 0)),
        pl.BlockSpec((BR, 1), lambda i: (i, 0)),
        pl.BlockSpec((D, D), lambda i: (0, 0)),
        pl.BlockSpec((1, D), lambda i: (0, 0)),
    ],
    out_specs=pl.BlockSpec((BR, D), lambda i: (i, 0)),
    out_shape=jax.ShapeDtypeStruct((O, D), jnp.float32),
)


def _pred_body(pred, y1, Wm, bm, out):
    i = pl.program_id(0)
    cself = jnp.where(i < NBMID, 0.5, 1.0)
    cy = jnp.where(i < NBMID, INV_SQRT2, 0.0)
    y = pred[...] * cself + y1[...] * cy
    out[...] = lax.dot_general(
        y, Wm[...], (((1,), (1,)), ((), ())),
        preferred_element_type=jnp.float32) + bm[...]


_pred_call = pl.pallas_call(
    _pred_body,
    grid=(NBP,),
    in_specs=[
        pl.BlockSpec((BR, D), lambda i: (i, 0)),
        pl.BlockSpec((BR, D), lambda i: (jnp.minimum(i + NBO, NBP - 1), 0)),
        pl.BlockSpec((D, D), lambda i: (0, 0)),
        pl.BlockSpec((1, D), lambda i: (0, 0)),
    ],
    out_specs=pl.BlockSpec((BR, D), lambda i: (i, 0)),
    out_shape=jax.ShapeDtypeStruct((T, D), jnp.float32),
)


@jax.jit
def kernel(obj_vecs, pred_vecs, edges, W, b):
    s3 = edges[:, 0].reshape(NW, NCH, CH)
    o3 = edges[:, 1].reshape(NW, NCH, CH)
    bm = b.reshape(1, D)

    hist = _hist_call(o3)
    scaled, scaled2, dinv, invdeg = _prep_call(hist, obj_vecs)
    y1 = _gather_call(scaled, s3)
    accp = _scatter_call(pred_vecs, scaled2, o3)
    out_obj = _obj_call(obj_vecs, y1, accp, dinv, invdeg, W, bm)
    out_pred = _pred_call(pred_vecs, y1, W, bm)
    return out_obj, out_pred


# speedup vs baseline: 41.9825x; 1.1517x over previous
import functools

import jax
import jax.numpy as jnp
from jax import lax
from jax.experimental import pallas as pl
from jax.experimental.pallas import tpu as pltpu
from jax.experimental.pallas import tpu_sc as plsc

O = 10000
T = 320000
D = 128
N = O + T

NC, NS = 2, 16
NW = NC * NS
EPW = T // NW
CH = 80
NCH = EPW // CH
OPAD = 10240
RPT = OPAD // NS
INV_SQRT2 = 0.7071067811865476
SQRT2 = 1.4142135623730951

BR = 1000
NBO = O // BR
NBP = T // BR
NBMID = (T - O) // BR

_MESH = plsc.VectorSubcoreMesh(core_axis_name="c", subcore_axis_name="s")


def _hist_body(o3, hist_out, hist_sh, idx_v, ones_v, zbuf):
    core = lax.axis_index("c")
    sub = lax.axis_index("s")
    wid = sub * NC + core

    def _z(i, carry):
        zbuf[pl.ds(i * 16, 16)] = jnp.zeros((16,), jnp.int32)
        return carry

    lax.fori_loop(0, RPT // 16, _z, 0)

    def _o(i, carry):
        ones_v[pl.ds(i * 16, 16)] = jnp.ones((16,), jnp.int32)
        return carry

    lax.fori_loop(0, CH // 16, _o, 0)

    pltpu.sync_copy(zbuf, hist_sh.at[pl.ds(sub * RPT, RPT)])
    plsc.subcore_barrier()

    pltpu.sync_copy(o3.at[wid], idx_v)

    def _step(c, carry):
        pltpu.sync_copy(ones_v, hist_sh.at[idx_v.at[c]], add=True)
        return carry

    lax.fori_loop(0, NCH, _step, 0)
    plsc.subcore_barrier()
    pltpu.sync_copy(
        hist_sh.at[pl.ds(sub * RPT, RPT)],
        hist_out.at[core, pl.ds(sub * RPT, RPT)],
    )


_hist_call = pl.kernel(
    _hist_body,
    out_type=jax.ShapeDtypeStruct((NC, OPAD), jnp.int32),
    mesh=_MESH,
    scratch_types=[
        pltpu.VMEM_SHARED((OPAD,), jnp.int32),
        pltpu.VMEM((NCH, CH), jnp.int32),
        pltpu.VMEM((CH,), jnp.int32),
        pltpu.VMEM((RPT,), jnp.int32),
    ],
)


def _prep_body(hist, obj, scaled, scaled2, dinv, invdeg):
    h = hist[...].astype(jnp.float32)
    ones = jnp.ones((NC, 1), jnp.float32)
    deg = lax.dot_general(h, ones, (((0,), (0,)), ((), ())),
                          preferred_element_type=jnp.float32) + 2.0
    dv = lax.rsqrt(deg)
    dinv[...] = dv
    invdeg[...] = 1.0 / deg
    s = obj[...] * dv
    scaled[...] = s
    scaled2[...] = s * SQRT2


BRK = 2048

_prep_call = pl.pallas_call(
    _prep_body,
    grid=(OPAD // BRK,),
    in_specs=[
        pl.BlockSpec((NC, BRK), lambda i: (0, i)),
        pl.BlockSpec((BRK, D), lambda i: (i, 0)),
    ],
    out_specs=[
        pl.BlockSpec((BRK, D), lambda i: (i, 0)),
        pl.BlockSpec((BRK, D), lambda i: (i, 0)),
        pl.BlockSpec((BRK, 1), lambda i: (i, 0)),
        pl.BlockSpec((BRK, 1), lambda i: (i, 0)),
    ],
    out_shape=[
        jax.ShapeDtypeStruct((O, D), jnp.float32),
        jax.ShapeDtypeStruct((O, D), jnp.float32),
        jax.ShapeDtypeStruct((O, 1), jnp.float32),
        jax.ShapeDtypeStruct((O, 1), jnp.float32),
    ],
)


def _gather_body(tab, s3, y1, idx_v, rows0, rows1, sem0, sem1):
    core = lax.axis_index("c")
    sub = lax.axis_index("s")
    wid = sub * NC + core
    base = wid * EPW
    pltpu.sync_copy(s3.at[wid], idx_v)

    pltpu.async_copy(tab.at[idx_v.at[0]], rows0, sem0)

    def _step(i, carry):
        c = 2 * i
        pltpu.async_copy(tab.at[idx_v.at[c + 1]], rows1, sem1)
        pltpu.make_async_copy(tab.at[idx_v.at[c]], rows0, sem0).wait()
        pltpu.sync_copy(rows0, y1.at[pl.ds(base + c * CH, CH)])
        pltpu.async_copy(tab.at[idx_v.at[c + 2]], rows0, sem0)
        pltpu.make_async_copy(tab.at[idx_v.at[c + 1]], rows1, sem1).wait()
        pltpu.sync_copy(rows1, y1.at[pl.ds(base + (c + 1) * CH, CH)])
        return carry

    lax.fori_loop(0, (NCH - 1) // 2, _step, 0)
    pltpu.make_async_copy(tab.at[idx_v.at[NCH - 1]], rows0, sem0).wait()
    pltpu.sync_copy(rows0, y1.at[pl.ds(base + (NCH - 1) * CH, CH)])


_gather_call = pl.kernel(
    _gather_body,
    out_type=jax.ShapeDtypeStruct((T, D), jnp.float32),
    mesh=_MESH,
    scratch_types=[
        pltpu.VMEM((NCH, CH), jnp.int32),
        pltpu.VMEM((CH, D), jnp.float32),
        pltpu.VMEM((CH, D), jnp.float32),
        pltpu.SemaphoreType.DMA,
        pltpu.SemaphoreType.DMA,
    ],
)


def _scatter_body(pred, scaled2, o3, acc_out, acc, idx_v, rows0, rows1, zbuf,
                  sem0, sem1):
    core = lax.axis_index("c")
    sub = lax.axis_index("s")
    wid = sub * NC + core

    def _z(i, carry):
        r = i // 8
        cc = (i % 8) * 16
        zbuf[r, pl.ds(cc, 16)] = jnp.zeros((16,), jnp.float32)
        return carry

    lax.fori_loop(0, CH * 8, _z, 0)
    for j in range(RPT // CH):
        pltpu.sync_copy(zbuf, acc.at[pl.ds(sub * RPT + j * CH, CH)])
    plsc.subcore_barrier()

    pltpu.sync_copy(o3.at[wid], idx_v)

    def _fire(c, rows, sem):
        @pl.when(wid == 0)
        def _():
            pltpu.async_copy(scaled2.at[pl.ds(c * CH, CH)], rows, sem)

        @pl.when(wid != 0)
        def _():
            pltpu.async_copy(pred.at[pl.ds(wid * EPW - O + c * CH, CH)],
                             rows, sem)

    def _drain(rows, sem):
        pltpu.make_async_copy(pred.at[pl.ds(0, CH)], rows, sem).wait()

    _fire(0, rows0, sem0)

    def _step(i, carry):
        c = 2 * i
        _fire(c + 1, rows1, sem1)
        _drain(rows0, sem0)
        pltpu.sync_copy(rows0, acc.at[idx_v.at[c]], add=True)
        _fire(c + 2, rows0, sem0)
        _drain(rows1, sem1)
        pltpu.sync_copy(rows1, acc.at[idx_v.at[c + 1]], add=True)
        return carry

    lax.fori_loop(0, (NCH - 1) // 2, _step, 0)
    _drain(rows0, sem0)
    pltpu.sync_copy(rows0, acc.at[idx_v.at[NCH - 1]], add=True)

    plsc.subcore_barrier()
    for j in range(RPT // CH):
        sl = pl.ds(sub * RPT + j * CH, CH)
        pltpu.sync_copy(acc.at[sl], acc_out.at[core, sl])


_scatter_call = pl.kernel(
    _scatter_body,
    out_type=jax.ShapeDtypeStruct((NC, OPAD, D), jnp.float32),
    mesh=_MESH,
    scratch_types=[
        pltpu.VMEM_SHARED((OPAD, D), jnp.float32),
        pltpu.VMEM((NCH, CH), jnp.int32),
        pltpu.VMEM((CH, D), jnp.float32),
        pltpu.VMEM((CH, D), jnp.float32),
        pltpu.VMEM((CH, D), jnp.float32),
        pltpu.SemaphoreType.DMA,
        pltpu.SemaphoreType.DMA,
    ],
)


def _obj_body(obj, y1, accs, dinv, invdeg, Wm, bm, out):
    a = accs[...]
    dv = dinv[...]
    y = (obj[...] * invdeg[...]
         + y1[...] * dv
         + (a[0] + a[1]) * (dv * INV_SQRT2))
    out[...] = lax.dot_general(
        y, Wm[...], (((1,), (1,)), ((), ())),
        preferred_element_type=jnp.float32) + bm[...]


_obj_call = pl.pallas_call(
    _obj_body,
    grid=(NBO,),
    in_specs=[
        pl.BlockSpec((BR, D), lambda i: (i, 0)),
        pl.BlockSpec((BR, D), lambda i: (i, 0)),
        pl.BlockSpec((NC, BR, D), lambda i: (0, i, 0)),
        pl.BlockSpec((BR, 1), lambda i: (i, 0)),
        pl.BlockSpec((BR, 1), lambda i: (i, 0)),
        pl.BlockSpec((D, D), lambda i: (0, 0)),
        pl.BlockSpec((1, D), lambda i: (0, 0)),
    ],
    out_specs=pl.BlockSpec((BR, D), lambda i: (i, 0)),
    out_shape=jax.ShapeDtypeStruct((O, D), jnp.float32),
)


def _pred_body(pred, y1, Wm, bm, out):
    i = pl.program_id(0)
    cself = jnp.where(i < NBMID, 0.5, 1.0)
    cy = jnp.where(i < NBMID, INV_SQRT2, 0.0)
    y = pred[...] * cself + y1[...] * cy
    out[...] = lax.dot_general(
        y, Wm[...], (((1,), (1,)), ((), ())),
        preferred_element_type=jnp.float32) + bm[...]


_pred_call = pl.pallas_call(
    _pred_body,
    grid=(NBP,),
    in_specs=[
        pl.BlockSpec((BR, D), lambda i: (i, 0)),
        pl.BlockSpec((BR, D), lambda i: (jnp.minimum(i + NBO, NBP - 1), 0)),
        pl.BlockSpec((D, D), lambda i: (0, 0)),
        pl.BlockSpec((1, D), lambda i: (0, 0)),
    ],
    out_specs=pl.BlockSpec((BR, D), lambda i: (i, 0)),
    out_shape=jax.ShapeDtypeStruct((T, D), jnp.float32),
)


@jax.jit
def kernel(obj_vecs, pred_vecs, edges, W, b):
    s3 = edges[:, 0].reshape(NW, NCH, CH)
    o3 = edges[:, 1].reshape(NW, NCH, CH)
    bm = b.reshape(1, D)

    hist = _hist_call(o3)
    scaled, scaled2, dinv, invdeg = _prep_call(hist, obj_vecs)
    y1 = _gather_call(scaled, s3)
    accp = _scatter_call(pred_vecs, scaled2, o3)
    out_obj = _obj_call(obj_vecs, y1, accp, dinv, invdeg, W, bm)
    out_pred = _pred_call(pred_vecs, y1, W, bm)
    return out_obj, out_pred


# speedup vs baseline: 42.0139x; 1.0007x over previous
import functools

import jax
import jax.numpy as jnp
from jax import lax
from jax.experimental import pallas as pl
from jax.experimental.pallas import tpu as pltpu
from jax.experimental.pallas import tpu_sc as plsc

O = 10000
T = 320000
D = 128
N = O + T

NC, NS = 2, 16
NW = NC * NS
EPW = T // NW
CH = 80
NCH = EPW // CH
OPAD = 10240
RPT = OPAD // NS
INV_SQRT2 = 0.7071067811865476
SQRT2 = 1.4142135623730951

BR = 1000
NBO = O // BR
NBP = T // BR
NBMID = (T - O) // BR

_MESH = plsc.VectorSubcoreMesh(core_axis_name="c", subcore_axis_name="s")


def _hist_body(o3, hist_out, hist_sh, idx_v, ones_v, zbuf):
    core = lax.axis_index("c")
    sub = lax.axis_index("s")
    wid = sub * NC + core

    def _z(i, carry):
        zbuf[pl.ds(i * 16, 16)] = jnp.zeros((16,), jnp.int32)
        return carry

    lax.fori_loop(0, RPT // 16, _z, 0)

    def _o(i, carry):
        ones_v[pl.ds(i * 16, 16)] = jnp.ones((16,), jnp.int32)
        return carry

    lax.fori_loop(0, CH // 16, _o, 0)

    pltpu.sync_copy(zbuf, hist_sh.at[pl.ds(sub * RPT, RPT)])
    plsc.subcore_barrier()

    pltpu.sync_copy(o3.at[wid], idx_v)

    def _step(c, carry):
        pltpu.sync_copy(ones_v, hist_sh.at[idx_v.at[c]], add=True)
        return carry

    lax.fori_loop(0, NCH, _step, 0)
    plsc.subcore_barrier()
    pltpu.sync_copy(
        hist_sh.at[pl.ds(sub * RPT, RPT)],
        hist_out.at[core, pl.ds(sub * RPT, RPT)],
    )


_hist_call = pl.kernel(
    _hist_body,
    out_type=jax.ShapeDtypeStruct((NC, OPAD), jnp.int32),
    mesh=_MESH,
    scratch_types=[
        pltpu.VMEM_SHARED((OPAD,), jnp.int32),
        pltpu.VMEM((NCH, CH), jnp.int32),
        pltpu.VMEM((CH,), jnp.int32),
        pltpu.VMEM((RPT,), jnp.int32),
    ],
)


def _prep_body(hist, obj, scaled, scaled2, dinv, invdeg):
    h = hist[...].astype(jnp.float32)
    ones = jnp.ones((NC, 1), jnp.float32)
    deg = lax.dot_general(h, ones, (((0,), (0,)), ((), ())),
                          preferred_element_type=jnp.float32) + 2.0
    dv = lax.rsqrt(deg)
    dinv[...] = dv
    invdeg[...] = 1.0 / deg
    s = obj[...] * dv
    scaled[...] = s
    scaled2[...] = s * SQRT2


BRK = 2048

_prep_call = pl.pallas_call(
    _prep_body,
    grid=(OPAD // BRK,),
    in_specs=[
        pl.BlockSpec((NC, BRK), lambda i: (0, i)),
        pl.BlockSpec((BRK, D), lambda i: (i, 0)),
    ],
    out_specs=[
        pl.BlockSpec((BRK, D), lambda i: (i, 0)),
        pl.BlockSpec((BRK, D), lambda i: (i, 0)),
        pl.BlockSpec((BRK, 1), lambda i: (i, 0)),
        pl.BlockSpec((BRK, 1), lambda i: (i, 0)),
    ],
    out_shape=[
        jax.ShapeDtypeStruct((O, D), jnp.float32),
        jax.ShapeDtypeStruct((O, D), jnp.float32),
        jax.ShapeDtypeStruct((O, 1), jnp.float32),
        jax.ShapeDtypeStruct((O, 1), jnp.float32),
    ],
)


def _gather_body(tab, s3, y1, idx_v, rows0, rows1, sem0, sem1):
    core = lax.axis_index("c")
    sub = lax.axis_index("s")
    wid = sub * NC + core
    base = wid * EPW
    pltpu.sync_copy(s3.at[wid], idx_v)

    pltpu.async_copy(tab.at[idx_v.at[0]], rows0, sem0)

    def _step(i, carry):
        c = 2 * i
        pltpu.async_copy(tab.at[idx_v.at[c + 1]], rows1, sem1)
        pltpu.make_async_copy(tab.at[idx_v.at[c]], rows0, sem0).wait()
        pltpu.sync_copy(rows0, y1.at[pl.ds(base + c * CH, CH)])
        pltpu.async_copy(tab.at[idx_v.at[c + 2]], rows0, sem0)
        pltpu.make_async_copy(tab.at[idx_v.at[c + 1]], rows1, sem1).wait()
        pltpu.sync_copy(rows1, y1.at[pl.ds(base + (c + 1) * CH, CH)])
        return carry

    lax.fori_loop(0, (NCH - 1) // 2, _step, 0)
    pltpu.make_async_copy(tab.at[idx_v.at[NCH - 1]], rows0, sem0).wait()
    pltpu.sync_copy(rows0, y1.at[pl.ds(base + (NCH - 1) * CH, CH)])


_gather_call = pl.kernel(
    _gather_body,
    out_type=jax.ShapeDtypeStruct((T, D), jnp.float32),
    mesh=_MESH,
    scratch_types=[
        pltpu.VMEM((NCH, CH), jnp.int32),
        pltpu.VMEM((CH, D), jnp.float32),
        pltpu.VMEM((CH, D), jnp.float32),
        pltpu.SemaphoreType.DMA,
        pltpu.SemaphoreType.DMA,
    ],
)


def _scatter_body(pred, scaled2, o3, acc_out, acc, idx_v, rows0, rows1, zbuf,
                  sem0, sem1):
    core = lax.axis_index("c")
    sub = lax.axis_index("s")
    wid = sub * NC + core

    def _z(i, carry):
        r = i // 8
        cc = (i % 8) * 16
        zbuf[r, pl.ds(cc, 16)] = jnp.zeros((16,), jnp.float32)
        return carry

    lax.fori_loop(0, CH * 8, _z, 0)
    for j in range(RPT // CH):
        pltpu.sync_copy(zbuf, acc.at[pl.ds(sub * RPT + j * CH, CH)])
    plsc.subcore_barrier()

    pltpu.sync_copy(o3.at[wid], idx_v)

    def _fire(c, rows, sem):
        @pl.when(wid == 0)
        def _():
            pltpu.async_copy(scaled2.at[pl.ds(c * CH, CH)], rows, sem)

        @pl.when(wid != 0)
        def _():
            pltpu.async_copy(pred.at[pl.ds(wid * EPW - O + c * CH, CH)],
                             rows, sem)

    def _drain(rows, sem):
        pltpu.make_async_copy(pred.at[pl.ds(0, CH)], rows, sem).wait()

    _fire(0, rows0, sem0)

    def _step(i, carry):
        c = 2 * i
        _fire(c + 1, rows1, sem1)
        _drain(rows0, sem0)
        pltpu.sync_copy(rows0, acc.at[idx_v.at[c]], add=True)
        _fire(c + 2, rows0, sem0)
        _drain(rows1, sem1)
        pltpu.sync_copy(rows1, acc.at[idx_v.at[c + 1]], add=True)
        return carry

    lax.fori_loop(0, (NCH - 1) // 2, _step, 0)
    _drain(rows0, sem0)
    pltpu.sync_copy(rows0, acc.at[idx_v.at[NCH - 1]], add=True)

    plsc.subcore_barrier()
    for j in range(RPT // CH):
        sl = pl.ds(sub * RPT + j * CH, CH)
        pltpu.sync_copy(acc.at[sl], acc_out.at[core, sl])


_scatter_call = pl.kernel(
    _scatter_body,
    out_type=jax.ShapeDtypeStruct((NC, OPAD, D), jnp.float32),
    mesh=_MESH,
    scratch_types=[
        pltpu.VMEM_SHARED((OPAD, D), jnp.float32),
        pltpu.VMEM((NCH, CH), jnp.int32),
        pltpu.VMEM((CH, D), jnp.float32),
        pltpu.VMEM((CH, D), jnp.float32),
        pltpu.VMEM((CH, D), jnp.float32),
        pltpu.SemaphoreType.DMA,
        pltpu.SemaphoreType.DMA,
    ],
)


def _obj_body(obj, y1, accs, dinv, invdeg, Wm, bm, out):
    a = accs[...]
    dv = dinv[...]
    y = (obj[...] * invdeg[...]
         + y1[...] * dv
         + (a[0] + a[1]) * (dv * INV_SQRT2))
    out[...] = lax.dot_general(
        y, Wm[...], (((1,), (1,)), ((), ())),
        preferred_element_type=jnp.float32) + bm[...]


_obj_call = pl.pallas_call(
    _obj_body,
    grid=(NBO,),
    in_specs=[
        pl.BlockSpec((BR, D), lambda i: (i, 0)),
        pl.BlockSpec((BR, D), lambda i: (i, 0)),
        pl.BlockSpec((NC, BR, D), lambda i: (0, i, 0)),
        pl.BlockSpec((BR, 1), lambda i: (i, 0)),
        pl.BlockSpec((BR, 1), lambda i: (i, 0)),
        pl.BlockSpec((D, D), lambda i: (0, 0)),
        pl.BlockSpec((1, D), lambda i: (0, 0)),
    ],
    out_specs=pl.BlockSpec((BR, D), lambda i: (i, 0)),
    out_shape=jax.ShapeDtypeStruct((O, D), jnp.float32),
)


def _pred_body(pred, y1, Wm, bm, out):
    i = pl.program_id(0)
    cself = jnp.where(i < NBMID, 0.5, 1.0)
    cy = jnp.where(i < NBMID, INV_SQRT2, 0.0)
    y = pred[...] * cself + y1[...] * cy
    out[...] = lax.dot_general(
        y, Wm[...], (((1,), (1,)), ((), ())),
        preferred_element_type=jnp.float32) + bm[...]


_pred_call = pl.pallas_call(
    _pred_body,
    grid=(NBP,),
    in_specs=[
        pl.BlockSpec((BR, D), lambda i: (i, 0)),
        pl.BlockSpec((BR, D), lambda i: (jnp.minimum(i + NBO, NBP - 1), 0)),
        pl.BlockSpec((D, D), lambda i: (0, 0)),
        pl.BlockSpec((1, D), lambda i: (0, 0)),
    ],
    out_specs=pl.BlockSpec((BR, D), lambda i: (i, 0)),
    out_shape=jax.ShapeDtypeStruct((T, D), jnp.float32),
)


@jax.jit
def kernel(obj_vecs, pred_vecs, edges, W, b):
    s3 = edges[:, 0].reshape(NW, NCH, CH)
    o3 = edges[:, 1].reshape(NW, NCH, CH)
    bm = b.reshape(1, D)

    hist = _hist_call(o3)
    scaled, scaled2, dinv, invdeg = _prep_call(hist, obj_vecs)
    y1 = _gather_call(scaled, s3)
    accp = _scatter_call(pred_vecs, scaled2, o3)
    out_pred = _pred_call(pred_vecs, y1, W, bm)
    out_obj = _obj_call(obj_vecs, y1, accp, dinv, invdeg, W, bm)
    return out_obj, out_pred


# speedup vs baseline: 50.6565x; 1.2057x over previous
import functools

import jax
import jax.numpy as jnp
from jax import lax
from jax.experimental import pallas as pl
from jax.experimental.pallas import tpu as pltpu
from jax.experimental.pallas import tpu_sc as plsc

O = 10000
T = 320000
D = 128
N = O + T

NC, NS = 2, 16
NW = NC * NS
EPW = T // NW
CH = 80
NCH = EPW // CH
OPAD = 10240
RPT = OPAD // NS
INV_SQRT2 = 0.7071067811865476
SQRT2 = 1.4142135623730951

BR = 2000
NBO = O // BR
NBP = T // BR
NBMID = (T - O) // BR

_MESH = plsc.VectorSubcoreMesh(core_axis_name="c", subcore_axis_name="s")


def _hist_body(o3, hist_out, hist_sh, idx_v, ones_v, zbuf):
    core = lax.axis_index("c")
    sub = lax.axis_index("s")
    wid = sub * NC + core

    def _z(i, carry):
        zbuf[pl.ds(i * 16, 16)] = jnp.zeros((16,), jnp.int32)
        return carry

    lax.fori_loop(0, RPT // 16, _z, 0)

    def _o(i, carry):
        ones_v[pl.ds(i * 16, 16)] = jnp.ones((16,), jnp.int32)
        return carry

    lax.fori_loop(0, CH // 16, _o, 0)

    pltpu.sync_copy(zbuf, hist_sh.at[pl.ds(sub * RPT, RPT)])
    plsc.subcore_barrier()

    pltpu.sync_copy(o3.at[wid], idx_v)

    def _step(c, carry):
        pltpu.sync_copy(ones_v, hist_sh.at[idx_v.at[c]], add=True)
        return carry

    lax.fori_loop(0, NCH, _step, 0)
    plsc.subcore_barrier()
    pltpu.sync_copy(
        hist_sh.at[pl.ds(sub * RPT, RPT)],
        hist_out.at[core, pl.ds(sub * RPT, RPT)],
    )


_hist_call = pl.kernel(
    _hist_body,
    out_type=jax.ShapeDtypeStruct((NC, OPAD), jnp.int32),
    mesh=_MESH,
    scratch_types=[
        pltpu.VMEM_SHARED((OPAD,), jnp.int32),
        pltpu.VMEM((NCH, CH), jnp.int32),
        pltpu.VMEM((CH,), jnp.int32),
        pltpu.VMEM((RPT,), jnp.int32),
    ],
)


def _prep_body(hist, obj, scaled, scaled2, dinv, invdeg):
    h = hist[...].astype(jnp.float32)
    ones = jnp.ones((NC, 1), jnp.float32)
    deg = lax.dot_general(h, ones, (((0,), (0,)), ((), ())),
                          preferred_element_type=jnp.float32) + 2.0
    dv = lax.rsqrt(deg)
    dinv[...] = dv
    invdeg[...] = 1.0 / deg
    s = obj[...] * dv
    scaled[...] = s
    scaled2[...] = s * SQRT2


BRK = 2048

_prep_call = pl.pallas_call(
    _prep_body,
    grid=(OPAD // BRK,),
    in_specs=[
        pl.BlockSpec((NC, BRK), lambda i: (0, i)),
        pl.BlockSpec((BRK, D), lambda i: (i, 0)),
    ],
    out_specs=[
        pl.BlockSpec((BRK, D), lambda i: (i, 0)),
        pl.BlockSpec((BRK, D), lambda i: (i, 0)),
        pl.BlockSpec((BRK, 1), lambda i: (i, 0)),
        pl.BlockSpec((BRK, 1), lambda i: (i, 0)),
    ],
    out_shape=[
        jax.ShapeDtypeStruct((O, D), jnp.float32),
        jax.ShapeDtypeStruct((O, D), jnp.float32),
        jax.ShapeDtypeStruct((O, 1), jnp.float32),
        jax.ShapeDtypeStruct((O, 1), jnp.float32),
    ],
)


def _gather_body(tab, s3, y1, idx_v, rows0, rows1, sem0, sem1):
    core = lax.axis_index("c")
    sub = lax.axis_index("s")
    wid = sub * NC + core
    base = wid * EPW
    pltpu.sync_copy(s3.at[wid], idx_v)

    pltpu.async_copy(tab.at[idx_v.at[0]], rows0, sem0)

    def _step(i, carry):
        c = 2 * i
        pltpu.async_copy(tab.at[idx_v.at[c + 1]], rows1, sem1)
        pltpu.make_async_copy(tab.at[idx_v.at[c]], rows0, sem0).wait()
        pltpu.sync_copy(rows0, y1.at[pl.ds(base + c * CH, CH)])
        pltpu.async_copy(tab.at[idx_v.at[c + 2]], rows0, sem0)
        pltpu.make_async_copy(tab.at[idx_v.at[c + 1]], rows1, sem1).wait()
        pltpu.sync_copy(rows1, y1.at[pl.ds(base + (c + 1) * CH, CH)])
        return carry

    lax.fori_loop(0, (NCH - 1) // 2, _step, 0)
    pltpu.make_async_copy(tab.at[idx_v.at[NCH - 1]], rows0, sem0).wait()
    pltpu.sync_copy(rows0, y1.at[pl.ds(base + (NCH - 1) * CH, CH)])


_gather_call = pl.kernel(
    _gather_body,
    out_type=jax.ShapeDtypeStruct((T, D), jnp.float32),
    mesh=_MESH,
    scratch_types=[
        pltpu.VMEM((NCH, CH), jnp.int32),
        pltpu.VMEM((CH, D), jnp.float32),
        pltpu.VMEM((CH, D), jnp.float32),
        pltpu.SemaphoreType.DMA,
        pltpu.SemaphoreType.DMA,
    ],
)


def _scatter_body(pred, scaled2, o3, acc_out, acc, idx_v, rows0, rows1, zbuf,
                  sem0, sem1):
    core = lax.axis_index("c")
    sub = lax.axis_index("s")
    wid = sub * NC + core

    def _z(i, carry):
        r = i // 8
        cc = (i % 8) * 16
        zbuf[r, pl.ds(cc, 16)] = jnp.zeros((16,), jnp.float32)
        return carry

    lax.fori_loop(0, CH * 8, _z, 0)
    for j in range(RPT // CH):
        pltpu.sync_copy(zbuf, acc.at[pl.ds(sub * RPT + j * CH, CH)])
    plsc.subcore_barrier()

    pltpu.sync_copy(o3.at[wid], idx_v)

    def _fire(c, rows, sem):
        @pl.when(wid == 0)
        def _():
            pltpu.async_copy(scaled2.at[pl.ds(c * CH, CH)], rows, sem)

        @pl.when(wid != 0)
        def _():
            pltpu.async_copy(pred.at[pl.ds(wid * EPW - O + c * CH, CH)],
                             rows, sem)

    def _drain(rows, sem):
        pltpu.make_async_copy(pred.at[pl.ds(0, CH)], rows, sem).wait()

    _fire(0, rows0, sem0)

    def _step(i, carry):
        c = 2 * i
        _fire(c + 1, rows1, sem1)
        _drain(rows0, sem0)
        pltpu.sync_copy(rows0, acc.at[idx_v.at[c]], add=True)
        _fire(c + 2, rows0, sem0)
        _drain(rows1, sem1)
        pltpu.sync_copy(rows1, acc.at[idx_v.at[c + 1]], add=True)
        return carry

    lax.fori_loop(0, (NCH - 1) // 2, _step, 0)
    _drain(rows0, sem0)
    pltpu.sync_copy(rows0, acc.at[idx_v.at[NCH - 1]], add=True)

    plsc.subcore_barrier()
    for j in range(RPT // CH):
        sl = pl.ds(sub * RPT + j * CH, CH)
        pltpu.sync_copy(acc.at[sl], acc_out.at[core, sl])


_scatter_call = pl.kernel(
    _scatter_body,
    out_type=jax.ShapeDtypeStruct((NC, OPAD, D), jnp.float32),
    mesh=_MESH,
    scratch_types=[
        pltpu.VMEM_SHARED((OPAD, D), jnp.float32),
        pltpu.VMEM((NCH, CH), jnp.int32),
        pltpu.VMEM((CH, D), jnp.float32),
        pltpu.VMEM((CH, D), jnp.float32),
        pltpu.VMEM((CH, D), jnp.float32),
        pltpu.SemaphoreType.DMA,
        pltpu.SemaphoreType.DMA,
    ],
)


def _obj_body(obj, y1, accs, dinv, invdeg, Wm, bm, out):
    a = accs[...]
    dv = dinv[...]
    y = (obj[...] * invdeg[...]
         + y1[...] * dv
         + (a[0] + a[1]) * (dv * INV_SQRT2))
    out[...] = lax.dot_general(
        y, Wm[...], (((1,), (1,)), ((), ())),
        preferred_element_type=jnp.float32) + bm[...]


_obj_call = pl.pallas_call(
    _obj_body,
    grid=(NBO,),
    in_specs=[
        pl.BlockSpec((BR, D), lambda i: (i, 0)),
        pl.BlockSpec((BR, D), lambda i: (i, 0)),
        pl.BlockSpec((NC, BR, D), lambda i: (0, i, 0)),
        pl.BlockSpec((BR, 1), lambda i: (i, 0)),
        pl.BlockSpec((BR, 1), lambda i: (i, 0)),
        pl.BlockSpec((D, D), lambda i: (0, 0)),
        pl.BlockSpec((1, D), lambda i: (0, 0)),
    ],
    out_specs=pl.BlockSpec((BR, D), lambda i: (i, 0)),
    out_shape=jax.ShapeDtypeStruct((O, D), jnp.float32),
)


def _pred_body(pred, y1, Wm, bm, out):
    i = pl.program_id(0)
    cself = jnp.where(i < NBMID, 0.5, 1.0)
    cy = jnp.where(i < NBMID, INV_SQRT2, 0.0)
    y = pred[...] * cself + y1[...] * cy
    out[...] = lax.dot_general(
        y, Wm[...], (((1,), (1,)), ((), ())),
        preferred_element_type=jnp.float32) + bm[...]


_pred_call = pl.pallas_call(
    _pred_body,
    grid=(NBP,),
    in_specs=[
        pl.BlockSpec((BR, D), lambda i: (i, 0)),
        pl.BlockSpec((BR, D), lambda i: (jnp.minimum(i + NBO, NBP - 1), 0)),
        pl.BlockSpec((D, D), lambda i: (0, 0)),
        pl.BlockSpec((1, D), lambda i: (0, 0)),
    ],
    out_specs=pl.BlockSpec((BR, D), lambda i: (i, 0)),
    out_shape=jax.ShapeDtypeStruct((T, D), jnp.float32),
)


@jax.jit
def kernel(obj_vecs, pred_vecs, edges, W, b):
    s3 = edges[:, 0].reshape(NW, NCH, CH)
    o3 = edges[:, 1].reshape(NW, NCH, CH)
    bm = b.reshape(1, D)

    hist = _hist_call(o3)
    scaled, scaled2, dinv, invdeg = _prep_call(hist, obj_vecs)
    y1 = _gather_call(scaled, s3)
    accp = _scatter_call(pred_vecs, scaled2, o3)
    out_pred = _pred_call(pred_vecs, y1, W, bm)
    out_obj = _obj_call(obj_vecs, y1, accp, dinv, invdeg, W, bm)
    return out_obj, out_pred


# speedup vs baseline: 53.1604x; 1.0494x over previous
import functools

import jax
import jax.numpy as jnp
from jax import lax
from jax.experimental import pallas as pl
from jax.experimental.pallas import tpu as pltpu
from jax.experimental.pallas import tpu_sc as plsc

O = 10000
T = 320000
D = 128
N = O + T

NC, NS = 2, 16
NW = NC * NS
EPW = T // NW
CH = 80
NCH = EPW // CH
OPAD = 10240
RPT = OPAD // NS
INV_SQRT2 = 0.7071067811865476
SQRT2 = 1.4142135623730951

BR = 5000
NBO = O // BR
NBP = T // BR
NBMID = (T - O) // BR

_MESH = plsc.VectorSubcoreMesh(core_axis_name="c", subcore_axis_name="s")


def _hist_body(o3, hist_out, hist_sh, idx_v, ones_v, zbuf):
    core = lax.axis_index("c")
    sub = lax.axis_index("s")
    wid = sub * NC + core

    def _z(i, carry):
        zbuf[pl.ds(i * 16, 16)] = jnp.zeros((16,), jnp.int32)
        return carry

    lax.fori_loop(0, RPT // 16, _z, 0)

    def _o(i, carry):
        ones_v[pl.ds(i * 16, 16)] = jnp.ones((16,), jnp.int32)
        return carry

    lax.fori_loop(0, CH // 16, _o, 0)

    pltpu.sync_copy(zbuf, hist_sh.at[pl.ds(sub * RPT, RPT)])
    plsc.subcore_barrier()

    pltpu.sync_copy(o3.at[wid], idx_v)

    def _step(c, carry):
        pltpu.sync_copy(ones_v, hist_sh.at[idx_v.at[c]], add=True)
        return carry

    lax.fori_loop(0, NCH, _step, 0)
    plsc.subcore_barrier()
    pltpu.sync_copy(
        hist_sh.at[pl.ds(sub * RPT, RPT)],
        hist_out.at[core, pl.ds(sub * RPT, RPT)],
    )


_hist_call = pl.kernel(
    _hist_body,
    out_type=jax.ShapeDtypeStruct((NC, OPAD), jnp.int32),
    mesh=_MESH,
    scratch_types=[
        pltpu.VMEM_SHARED((OPAD,), jnp.int32),
        pltpu.VMEM((NCH, CH), jnp.int32),
        pltpu.VMEM((CH,), jnp.int32),
        pltpu.VMEM((RPT,), jnp.int32),
    ],
)


def _prep_body(hist, obj, scaled, scaled2, dinv, invdeg):
    h = hist[...].astype(jnp.float32)
    ones = jnp.ones((NC, 1), jnp.float32)
    deg = lax.dot_general(h, ones, (((0,), (0,)), ((), ())),
                          preferred_element_type=jnp.float32) + 2.0
    dv = lax.rsqrt(deg)
    dinv[...] = dv
    invdeg[...] = 1.0 / deg
    s = obj[...] * dv
    scaled[...] = s
    scaled2[...] = s * SQRT2


BRK = 2048

_prep_call = pl.pallas_call(
    _prep_body,
    grid=(OPAD // BRK,),
    in_specs=[
        pl.BlockSpec((NC, BRK), lambda i: (0, i)),
        pl.BlockSpec((BRK, D), lambda i: (i, 0)),
    ],
    out_specs=[
        pl.BlockSpec((BRK, D), lambda i: (i, 0)),
        pl.BlockSpec((BRK, D), lambda i: (i, 0)),
        pl.BlockSpec((BRK, 1), lambda i: (i, 0)),
        pl.BlockSpec((BRK, 1), lambda i: (i, 0)),
    ],
    out_shape=[
        jax.ShapeDtypeStruct((O, D), jnp.float32),
        jax.ShapeDtypeStruct((O, D), jnp.float32),
        jax.ShapeDtypeStruct((O, 1), jnp.float32),
        jax.ShapeDtypeStruct((O, 1), jnp.float32),
    ],
)


def _gather_body(tab, s3, y1, idx_v, rows0, rows1, sem0, sem1):
    core = lax.axis_index("c")
    sub = lax.axis_index("s")
    wid = sub * NC + core
    base = wid * EPW
    pltpu.sync_copy(s3.at[wid], idx_v)

    pltpu.async_copy(tab.at[idx_v.at[0]], rows0, sem0)

    def _step(i, carry):
        c = 2 * i
        pltpu.async_copy(tab.at[idx_v.at[c + 1]], rows1, sem1)
        pltpu.make_async_copy(tab.at[idx_v.at[c]], rows0, sem0).wait()
        pltpu.sync_copy(rows0, y1.at[pl.ds(base + c * CH, CH)])
        pltpu.async_copy(tab.at[idx_v.at[c + 2]], rows0, sem0)
        pltpu.make_async_copy(tab.at[idx_v.at[c + 1]], rows1, sem1).wait()
        pltpu.sync_copy(rows1, y1.at[pl.ds(base + (c + 1) * CH, CH)])
        return carry

    lax.fori_loop(0, (NCH - 1) // 2, _step, 0)
    pltpu.make_async_copy(tab.at[idx_v.at[NCH - 1]], rows0, sem0).wait()
    pltpu.sync_copy(rows0, y1.at[pl.ds(base + (NCH - 1) * CH, CH)])


_gather_call = pl.kernel(
    _gather_body,
    out_type=jax.ShapeDtypeStruct((T, D), jnp.float32),
    mesh=_MESH,
    scratch_types=[
        pltpu.VMEM((NCH, CH), jnp.int32),
        pltpu.VMEM((CH, D), jnp.float32),
        pltpu.VMEM((CH, D), jnp.float32),
        pltpu.SemaphoreType.DMA,
        pltpu.SemaphoreType.DMA,
    ],
)


def _scatter_body(pred, scaled2, o3, acc_out, acc, idx_v, rows0, rows1, zbuf,
                  sem0, sem1):
    core = lax.axis_index("c")
    sub = lax.axis_index("s")
    wid = sub * NC + core

    def _z(i, carry):
        r = i // 8
        cc = (i % 8) * 16
        zbuf[r, pl.ds(cc, 16)] = jnp.zeros((16,), jnp.float32)
        return carry

    lax.fori_loop(0, CH * 8, _z, 0)
    for j in range(RPT // CH):
        pltpu.sync_copy(zbuf, acc.at[pl.ds(sub * RPT + j * CH, CH)])
    plsc.subcore_barrier()

    pltpu.sync_copy(o3.at[wid], idx_v)

    def _fire(c, rows, sem):
        @pl.when(wid == 0)
        def _():
            pltpu.async_copy(scaled2.at[pl.ds(c * CH, CH)], rows, sem)

        @pl.when(wid != 0)
        def _():
            pltpu.async_copy(pred.at[pl.ds(wid * EPW - O + c * CH, CH)],
                             rows, sem)

    def _drain(rows, sem):
        pltpu.make_async_copy(pred.at[pl.ds(0, CH)], rows, sem).wait()

    _fire(0, rows0, sem0)

    def _step(i, carry):
        c = 2 * i
        _fire(c + 1, rows1, sem1)
        _drain(rows0, sem0)
        pltpu.sync_copy(rows0, acc.at[idx_v.at[c]], add=True)
        _fire(c + 2, rows0, sem0)
        _drain(rows1, sem1)
        pltpu.sync_copy(rows1, acc.at[idx_v.at[c + 1]], add=True)
        return carry

    lax.fori_loop(0, (NCH - 1) // 2, _step, 0)
    _drain(rows0, sem0)
    pltpu.sync_copy(rows0, acc.at[idx_v.at[NCH - 1]], add=True)

    plsc.subcore_barrier()
    for j in range(RPT // CH):
        sl = pl.ds(sub * RPT + j * CH, CH)
        pltpu.sync_copy(acc.at[sl], acc_out.at[core, sl])


_scatter_call = pl.kernel(
    _scatter_body,
    out_type=jax.ShapeDtypeStruct((NC, OPAD, D), jnp.float32),
    mesh=_MESH,
    scratch_types=[
        pltpu.VMEM_SHARED((OPAD, D), jnp.float32),
        pltpu.VMEM((NCH, CH), jnp.int32),
        pltpu.VMEM((CH, D), jnp.float32),
        pltpu.VMEM((CH, D), jnp.float32),
        pltpu.VMEM((CH, D), jnp.float32),
        pltpu.SemaphoreType.DMA,
        pltpu.SemaphoreType.DMA,
    ],
)


def _obj_body(obj, y1, accs, dinv, invdeg, Wm, bm, out):
    a = accs[...]
    dv = dinv[...]
    y = (obj[...] * invdeg[...]
         + y1[...] * dv
         + (a[0] + a[1]) * (dv * INV_SQRT2))
    out[...] = lax.dot_general(
        y, Wm[...], (((1,), (1,)), ((), ())),
        preferred_element_type=jnp.float32) + bm[...]


_obj_call = pl.pallas_call(
    _obj_body,
    grid=(NBO,),
    in_specs=[
        pl.BlockSpec((BR, D), lambda i: (i, 0)),
        pl.BlockSpec((BR, D), lambda i: (i, 0)),
        pl.BlockSpec((NC, BR, D), lambda i: (0, i, 0)),
        pl.BlockSpec((BR, 1), lambda i: (i, 0)),
        pl.BlockSpec((BR, 1), lambda i: (i, 0)),
        pl.BlockSpec((D, D), lambda i: (0, 0)),
        pl.BlockSpec((1, D), lambda i: (0, 0)),
    ],
    out_specs=pl.BlockSpec((BR, D), lambda i: (i, 0)),
    out_shape=jax.ShapeDtypeStruct((O, D), jnp.float32),
)


def _pred_body(pred, y1, Wm, bm, out):
    i = pl.program_id(0)
    cself = jnp.where(i < NBMID, 0.5, 1.0)
    cy = jnp.where(i < NBMID, INV_SQRT2, 0.0)
    y = pred[...] * cself + y1[...] * cy
    out[...] = lax.dot_general(
        y, Wm[...], (((1,), (1,)), ((), ())),
        preferred_element_type=jnp.float32) + bm[...]


_pred_call = pl.pallas_call(
    _pred_body,
    grid=(NBP,),
    in_specs=[
        pl.BlockSpec((BR, D), lambda i: (i, 0)),
        pl.BlockSpec((BR, D), lambda i: (jnp.minimum(i + NBO, NBP - 1), 0)),
        pl.BlockSpec((D, D), lambda i: (0, 0)),
        pl.BlockSpec((1, D), lambda i: (0, 0)),
    ],
    out_specs=pl.BlockSpec((BR, D), lambda i: (i, 0)),
    out_shape=jax.ShapeDtypeStruct((T, D), jnp.float32),
)


@jax.jit
def kernel(obj_vecs, pred_vecs, edges, W, b):
    s3 = edges[:, 0].reshape(NW, NCH, CH)
    o3 = edges[:, 1].reshape(NW, NCH, CH)
    bm = b.reshape(1, D)

    hist = _hist_call(o3)
    scaled, scaled2, dinv, invdeg = _prep_call(hist, obj_vecs)
    y1 = _gather_call(scaled, s3)
    accp = _scatter_call(pred_vecs, scaled2, o3)
    out_pred = _pred_call(pred_vecs, y1, W, bm)
    out_obj = _obj_call(obj_vecs, y1, accp, dinv, invdeg, W, bm)
    return out_obj, out_pred


# speedup vs baseline: 53.4151x; 1.0048x over previous
import functools

import jax
import jax.numpy as jnp
from jax import lax
from jax.experimental import pallas as pl
from jax.experimental.pallas import tpu as pltpu
from jax.experimental.pallas import tpu_sc as plsc

O = 10000
T = 320000
D = 128
N = O + T

NC, NS = 2, 16
NW = NC * NS
EPW = T // NW
CH = 80
NCH = EPW // CH
OPAD = 10240
RPT = OPAD // NS
INV_SQRT2 = 0.7071067811865476
SQRT2 = 1.4142135623730951

BR = 10000
NBO = O // BR
NBP = T // BR
NBMID = (T - O) // BR

_MESH = plsc.VectorSubcoreMesh(core_axis_name="c", subcore_axis_name="s")


def _hist_body(o3, hist_out, hist_sh, idx_v, ones_v, zbuf):
    core = lax.axis_index("c")
    sub = lax.axis_index("s")
    wid = sub * NC + core

    def _z(i, carry):
        zbuf[pl.ds(i * 16, 16)] = jnp.zeros((16,), jnp.int32)
        return carry

    lax.fori_loop(0, RPT // 16, _z, 0)

    def _o(i, carry):
        ones_v[pl.ds(i * 16, 16)] = jnp.ones((16,), jnp.int32)
        return carry

    lax.fori_loop(0, CH // 16, _o, 0)

    pltpu.sync_copy(zbuf, hist_sh.at[pl.ds(sub * RPT, RPT)])
    plsc.subcore_barrier()

    pltpu.sync_copy(o3.at[wid], idx_v)

    def _step(c, carry):
        pltpu.sync_copy(ones_v, hist_sh.at[idx_v.at[c]], add=True)
        return carry

    lax.fori_loop(0, NCH, _step, 0)
    plsc.subcore_barrier()
    pltpu.sync_copy(
        hist_sh.at[pl.ds(sub * RPT, RPT)],
        hist_out.at[core, pl.ds(sub * RPT, RPT)],
    )


_hist_call = pl.kernel(
    _hist_body,
    out_type=jax.ShapeDtypeStruct((NC, OPAD), jnp.int32),
    mesh=_MESH,
    scratch_types=[
        pltpu.VMEM_SHARED((OPAD,), jnp.int32),
        pltpu.VMEM((NCH, CH), jnp.int32),
        pltpu.VMEM((CH,), jnp.int32),
        pltpu.VMEM((RPT,), jnp.int32),
    ],
)


def _prep_body(hist, obj, scaled, scaled2, dinv, invdeg):
    h = hist[...].astype(jnp.float32)
    ones = jnp.ones((NC, 1), jnp.float32)
    deg = lax.dot_general(h, ones, (((0,), (0,)), ((), ())),
                          preferred_element_type=jnp.float32) + 2.0
    dv = lax.rsqrt(deg)
    dinv[...] = dv
    invdeg[...] = 1.0 / deg
    s = obj[...] * dv
    scaled[...] = s
    scaled2[...] = s * SQRT2


BRK = 2048

_prep_call = pl.pallas_call(
    _prep_body,
    grid=(OPAD // BRK,),
    in_specs=[
        pl.BlockSpec((NC, BRK), lambda i: (0, i)),
        pl.BlockSpec((BRK, D), lambda i: (i, 0)),
    ],
    out_specs=[
        pl.BlockSpec((BRK, D), lambda i: (i, 0)),
        pl.BlockSpec((BRK, D), lambda i: (i, 0)),
        pl.BlockSpec((BRK, 1), lambda i: (i, 0)),
        pl.BlockSpec((BRK, 1), lambda i: (i, 0)),
    ],
    out_shape=[
        jax.ShapeDtypeStruct((O, D), jnp.float32),
        jax.ShapeDtypeStruct((O, D), jnp.float32),
        jax.ShapeDtypeStruct((O, 1), jnp.float32),
        jax.ShapeDtypeStruct((O, 1), jnp.float32),
    ],
)


def _gather_body(tab, s3, y1, idx_v, rows0, rows1, sem0, sem1):
    core = lax.axis_index("c")
    sub = lax.axis_index("s")
    wid = sub * NC + core
    base = wid * EPW
    pltpu.sync_copy(s3.at[wid], idx_v)

    pltpu.async_copy(tab.at[idx_v.at[0]], rows0, sem0)

    def _step(i, carry):
        c = 2 * i
        pltpu.async_copy(tab.at[idx_v.at[c + 1]], rows1, sem1)
        pltpu.make_async_copy(tab.at[idx_v.at[c]], rows0, sem0).wait()
        pltpu.sync_copy(rows0, y1.at[pl.ds(base + c * CH, CH)])
        pltpu.async_copy(tab.at[idx_v.at[c + 2]], rows0, sem0)
        pltpu.make_async_copy(tab.at[idx_v.at[c + 1]], rows1, sem1).wait()
        pltpu.sync_copy(rows1, y1.at[pl.ds(base + (c + 1) * CH, CH)])
        return carry

    lax.fori_loop(0, (NCH - 1) // 2, _step, 0)
    pltpu.make_async_copy(tab.at[idx_v.at[NCH - 1]], rows0, sem0).wait()
    pltpu.sync_copy(rows0, y1.at[pl.ds(base + (NCH - 1) * CH, CH)])


_gather_call = pl.kernel(
    _gather_body,
    out_type=jax.ShapeDtypeStruct((T, D), jnp.float32),
    mesh=_MESH,
    scratch_types=[
        pltpu.VMEM((NCH, CH), jnp.int32),
        pltpu.VMEM((CH, D), jnp.float32),
        pltpu.VMEM((CH, D), jnp.float32),
        pltpu.SemaphoreType.DMA,
        pltpu.SemaphoreType.DMA,
    ],
)


def _scatter_body(pred, scaled2, o3, acc_out, acc, idx_v, rows0, rows1, zbuf,
                  sem0, sem1):
    core = lax.axis_index("c")
    sub = lax.axis_index("s")
    wid = sub * NC + core

    def _z(i, carry):
        r = i // 8
        cc = (i % 8) * 16
        zbuf[r, pl.ds(cc, 16)] = jnp.zeros((16,), jnp.float32)
        return carry

    lax.fori_loop(0, CH * 8, _z, 0)
    for j in range(RPT // CH):
        pltpu.sync_copy(zbuf, acc.at[pl.ds(sub * RPT + j * CH, CH)])
    plsc.subcore_barrier()

    pltpu.sync_copy(o3.at[wid], idx_v)

    def _fire(c, rows, sem):
        @pl.when(wid == 0)
        def _():
            pltpu.async_copy(scaled2.at[pl.ds(c * CH, CH)], rows, sem)

        @pl.when(wid != 0)
        def _():
            pltpu.async_copy(pred.at[pl.ds(wid * EPW - O + c * CH, CH)],
                             rows, sem)

    def _drain(rows, sem):
        pltpu.make_async_copy(pred.at[pl.ds(0, CH)], rows, sem).wait()

    _fire(0, rows0, sem0)

    def _step(i, carry):
        c = 2 * i
        _fire(c + 1, rows1, sem1)
        _drain(rows0, sem0)
        pltpu.sync_copy(rows0, acc.at[idx_v.at[c]], add=True)
        _fire(c + 2, rows0, sem0)
        _drain(rows1, sem1)
        pltpu.sync_copy(rows1, acc.at[idx_v.at[c + 1]], add=True)
        return carry

    lax.fori_loop(0, (NCH - 1) // 2, _step, 0)
    _drain(rows0, sem0)
    pltpu.sync_copy(rows0, acc.at[idx_v.at[NCH - 1]], add=True)

    plsc.subcore_barrier()
    for j in range(RPT // CH):
        sl = pl.ds(sub * RPT + j * CH, CH)
        pltpu.sync_copy(acc.at[sl], acc_out.at[core, sl])


_scatter_call = pl.kernel(
    _scatter_body,
    out_type=jax.ShapeDtypeStruct((NC, OPAD, D), jnp.float32),
    mesh=_MESH,
    scratch_types=[
        pltpu.VMEM_SHARED((OPAD, D), jnp.float32),
        pltpu.VMEM((NCH, CH), jnp.int32),
        pltpu.VMEM((CH, D), jnp.float32),
        pltpu.VMEM((CH, D), jnp.float32),
        pltpu.VMEM((CH, D), jnp.float32),
        pltpu.SemaphoreType.DMA,
        pltpu.SemaphoreType.DMA,
    ],
)


def _obj_body(obj, y1, accs, dinv, invdeg, Wm, bm, out):
    a = accs[...]
    dv = dinv[...]
    y = (obj[...] * invdeg[...]
         + y1[...] * dv
         + (a[0] + a[1]) * (dv * INV_SQRT2))
    out[...] = lax.dot_general(
        y, Wm[...], (((1,), (1,)), ((), ())),
        preferred_element_type=jnp.float32) + bm[...]


_obj_call = pl.pallas_call(
    _obj_body,
    grid=(NBO,),
    in_specs=[
        pl.BlockSpec((BR, D), lambda i: (i, 0)),
        pl.BlockSpec((BR, D), lambda i: (i, 0)),
        pl.BlockSpec((NC, BR, D), lambda i: (0, i, 0)),
        pl.BlockSpec((BR, 1), lambda i: (i, 0)),
        pl.BlockSpec((BR, 1), lambda i: (i, 0)),
        pl.BlockSpec((D, D), lambda i: (0, 0)),
        pl.BlockSpec((1, D), lambda i: (0, 0)),
    ],
    out_specs=pl.BlockSpec((BR, D), lambda i: (i, 0)),
    out_shape=jax.ShapeDtypeStruct((O, D), jnp.float32),
)


def _pred_body(pred, y1, Wm, bm, out):
    i = pl.program_id(0)
    cself = jnp.where(i < NBMID, 0.5, 1.0)
    cy = jnp.where(i < NBMID, INV_SQRT2, 0.0)
    y = pred[...] * cself + y1[...] * cy
    out[...] = lax.dot_general(
        y, Wm[...], (((1,), (1,)), ((), ())),
        preferred_element_type=jnp.float32) + bm[...]


_pred_call = pl.pallas_call(
    _pred_body,
    grid=(NBP,),
    in_specs=[
        pl.BlockSpec((BR, D), lambda i: (i, 0)),
        pl.BlockSpec((BR, D), lambda i: (jnp.minimum(i + NBO, NBP - 1), 0)),
        pl.BlockSpec((D, D), lambda i: (0, 0)),
        pl.BlockSpec((1, D), lambda i: (0, 0)),
    ],
    out_specs=pl.BlockSpec((BR, D), lambda i: (i, 0)),
    out_shape=jax.ShapeDtypeStruct((T, D), jnp.float32),
)


@jax.jit
def kernel(obj_vecs, pred_vecs, edges, W, b):
    s3 = edges[:, 0].reshape(NW, NCH, CH)
    o3 = edges[:, 1].reshape(NW, NCH, CH)
    bm = b.reshape(1, D)

    hist = _hist_call(o3)
    scaled, scaled2, dinv, invdeg = _prep_call(hist, obj_vecs)
    y1 = _gather_call(scaled, s3)
    accp = _scatter_call(pred_vecs, scaled2, o3)
    out_pred = _pred_call(pred_vecs, y1, W, bm)
    out_obj = _obj_call(obj_vecs, y1, accp, dinv, invdeg, W, bm)
    return out_obj, out_pred


# speedup vs baseline: 55.0943x; 1.0314x over previous
import functools

import jax
import jax.numpy as jnp
from jax import lax
from jax.experimental import pallas as pl
from jax.experimental.pallas import tpu as pltpu
from jax.experimental.pallas import tpu_sc as plsc

O = 10000
T = 320000
D = 128
N = O + T

NC, NS = 2, 16
NW = NC * NS
EPW = T // NW
CH = 80
NCH = EPW // CH
OPAD = 10240
RPT = OPAD // NS
INV_SQRT2 = 0.7071067811865476
SQRT2 = 1.4142135623730951

BR = 10000
NBO = O // BR
NBP = T // BR
NBMID = (T - O) // BR

_MESH = plsc.VectorSubcoreMesh(core_axis_name="c", subcore_axis_name="s")


def _hist_body(o3, hist_out, hist_sh, idx_v, ones_v, zbuf):
    core = lax.axis_index("c")
    sub = lax.axis_index("s")
    wid = sub * NC + core

    def _z(i, carry):
        zbuf[pl.ds(i * 16, 16)] = jnp.zeros((16,), jnp.int32)
        return carry

    lax.fori_loop(0, RPT // 16, _z, 0)

    def _o(i, carry):
        ones_v[pl.ds(i * 16, 16)] = jnp.ones((16,), jnp.int32)
        return carry

    lax.fori_loop(0, CH // 16, _o, 0)

    pltpu.sync_copy(zbuf, hist_sh.at[pl.ds(sub * RPT, RPT)])
    plsc.subcore_barrier()

    pltpu.sync_copy(o3.at[wid], idx_v)

    def _step(c, carry):
        pltpu.sync_copy(ones_v, hist_sh.at[idx_v.at[c]], add=True)
        return carry

    lax.fori_loop(0, NCH, _step, 0)
    plsc.subcore_barrier()
    pltpu.sync_copy(
        hist_sh.at[pl.ds(sub * RPT, RPT)],
        hist_out.at[core, pl.ds(sub * RPT, RPT)],
    )


_hist_call = pl.kernel(
    _hist_body,
    out_type=jax.ShapeDtypeStruct((NC, OPAD), jnp.int32),
    mesh=_MESH,
    scratch_types=[
        pltpu.VMEM_SHARED((OPAD,), jnp.int32),
        pltpu.VMEM((NCH, CH), jnp.int32),
        pltpu.VMEM((CH,), jnp.int32),
        pltpu.VMEM((RPT,), jnp.int32),
    ],
)


def _prep_body(hist, obj, scaled, scaled2, dinv, invdeg):
    h = hist[...].astype(jnp.float32)
    ones = jnp.ones((NC, 1), jnp.float32)
    deg = lax.dot_general(h, ones, (((0,), (0,)), ((), ())),
                          preferred_element_type=jnp.float32) + 2.0
    dv = lax.rsqrt(deg)
    dinv[...] = dv
    invdeg[...] = 1.0 / deg
    s = obj[...] * dv
    scaled[...] = s
    scaled2[...] = s * SQRT2


BRK = 2048

_prep_call = pl.pallas_call(
    _prep_body,
    grid=(OPAD // BRK,),
    in_specs=[
        pl.BlockSpec((NC, BRK), lambda i: (0, i)),
        pl.BlockSpec((BRK, D), lambda i: (i, 0)),
    ],
    out_specs=[
        pl.BlockSpec((BRK, D), lambda i: (i, 0)),
        pl.BlockSpec((BRK, D), lambda i: (i, 0)),
        pl.BlockSpec((BRK, 1), lambda i: (i, 0)),
        pl.BlockSpec((BRK, 1), lambda i: (i, 0)),
    ],
    out_shape=[
        jax.ShapeDtypeStruct((O, D), jnp.float32),
        jax.ShapeDtypeStruct((O, D), jnp.float32),
        jax.ShapeDtypeStruct((O, 1), jnp.float32),
        jax.ShapeDtypeStruct((O, 1), jnp.float32),
    ],
)


def _gather_body(tab, s3, y1, idx_v, rows0, rows1, rows2, sem0, sem1, sem2):
    core = lax.axis_index("c")
    sub = lax.axis_index("s")
    wid = sub * NC + core
    base = wid * EPW
    pltpu.sync_copy(s3.at[wid], idx_v)

    bufs = (rows0, sem0), (rows1, sem1), (rows2, sem2)

    def _fire(c, b):
        pltpu.async_copy(tab.at[idx_v.at[c]], b[0], b[1])

    def _put(c, b):
        pltpu.make_async_copy(tab.at[idx_v.at[c]], b[0], b[1]).wait()
        pltpu.sync_copy(b[0], y1.at[pl.ds(base + c * CH, CH)])

    _fire(0, bufs[0])
    _fire(1, bufs[1])

    def _step(i, carry):
        c = 3 * i
        _fire(c + 2, bufs[2])
        _put(c, bufs[0])
        _fire(c + 3, bufs[0])
        _put(c + 1, bufs[1])
        _fire(c + 4, bufs[1])
        _put(c + 2, bufs[2])
        return carry

    lax.fori_loop(0, (NCH - 2) // 3, _step, 0)
    _put(NCH - 2, bufs[0])
    _put(NCH - 1, bufs[1])


_gather_call = pl.kernel(
    _gather_body,
    out_type=jax.ShapeDtypeStruct((T, D), jnp.float32),
    mesh=_MESH,
    scratch_types=[
        pltpu.VMEM((NCH, CH), jnp.int32),
        pltpu.VMEM((CH, D), jnp.float32),
        pltpu.VMEM((CH, D), jnp.float32),
        pltpu.VMEM((CH, D), jnp.float32),
        pltpu.SemaphoreType.DMA,
        pltpu.SemaphoreType.DMA,
        pltpu.SemaphoreType.DMA,
    ],
)


def _scatter_body(pred, scaled2, o3, acc_out, acc, idx_v, rows0, rows1, rows2,
                  sem0, sem1, sem2):
    core = lax.axis_index("c")
    sub = lax.axis_index("s")
    wid = sub * NC + core

    def _z(i, carry):
        r = i // 8
        cc = (i % 8) * 16
        rows2[r, pl.ds(cc, 16)] = jnp.zeros((16,), jnp.float32)
        return carry

    lax.fori_loop(0, CH * 8, _z, 0)
    for j in range(RPT // CH):
        pltpu.sync_copy(rows2, acc.at[pl.ds(sub * RPT + j * CH, CH)])
    plsc.subcore_barrier()

    pltpu.sync_copy(o3.at[wid], idx_v)

    def _fire(c, rows, sem):
        @pl.when(wid == 0)
        def _():
            pltpu.async_copy(scaled2.at[pl.ds(c * CH, CH)], rows, sem)

        @pl.when(wid != 0)
        def _():
            pltpu.async_copy(pred.at[pl.ds(wid * EPW - O + c * CH, CH)],
                             rows, sem)

    def _drain(rows, sem):
        pltpu.make_async_copy(pred.at[pl.ds(0, CH)], rows, sem).wait()

    def _put(c, rows, sem):
        _drain(rows, sem)
        pltpu.sync_copy(rows, acc.at[idx_v.at[c]], add=True)

    _fire(0, rows0, sem0)
    _fire(1, rows1, sem1)

    def _step(i, carry):
        c = 3 * i
        _fire(c + 2, rows2, sem2)
        _put(c, rows0, sem0)
        _fire(c + 3, rows0, sem0)
        _put(c + 1, rows1, sem1)
        _fire(c + 4, rows1, sem1)
        _put(c + 2, rows2, sem2)
        return carry

    lax.fori_loop(0, (NCH - 2) // 3, _step, 0)
    _put(NCH - 2, rows0, sem0)
    _put(NCH - 1, rows1, sem1)

    plsc.subcore_barrier()
    for j in range(RPT // CH):
        sl = pl.ds(sub * RPT + j * CH, CH)
        pltpu.sync_copy(acc.at[sl], acc_out.at[core, sl])


_scatter_call = pl.kernel(
    _scatter_body,
    out_type=jax.ShapeDtypeStruct((NC, OPAD, D), jnp.float32),
    mesh=_MESH,
    scratch_types=[
        pltpu.VMEM_SHARED((OPAD, D), jnp.float32),
        pltpu.VMEM((NCH, CH), jnp.int32),
        pltpu.VMEM((CH, D), jnp.float32),
        pltpu.VMEM((CH, D), jnp.float32),
        pltpu.VMEM((CH, D), jnp.float32),
        pltpu.SemaphoreType.DMA,
        pltpu.SemaphoreType.DMA,
        pltpu.SemaphoreType.DMA,
    ],
)


def _obj_body(obj, y1, accs, dinv, invdeg, Wm, bm, out):
    a = accs[...]
    dv = dinv[...]
    y = (obj[...] * invdeg[...]
         + y1[...] * dv
         + (a[0] + a[1]) * (dv * INV_SQRT2))
    out[...] = lax.dot_general(
        y, Wm[...], (((1,), (1,)), ((), ())),
        preferred_element_type=jnp.float32) + bm[...]


_obj_call = pl.pallas_call(
    _obj_body,
    grid=(NBO,),
    in_specs=[
        pl.BlockSpec((BR, D), lambda i: (i, 0)),
        pl.BlockSpec((BR, D), lambda i: (i, 0)),
        pl.BlockSpec((NC, BR, D), lambda i: (0, i, 0)),
        pl.BlockSpec((BR, 1), lambda i: (i, 0)),
        pl.BlockSpec((BR, 1), lambda i: (i, 0)),
        pl.BlockSpec((D, D), lambda i: (0, 0)),
        pl.BlockSpec((1, D), lambda i: (0, 0)),
    ],
    out_specs=pl.BlockSpec((BR, D), lambda i: (i, 0)),
    out_shape=jax.ShapeDtypeStruct((O, D), jnp.float32),
)


def _pred_body(pred, y1, Wm, bm, out):
    i = pl.program_id(0)
    cself = jnp.where(i < NBMID, 0.5, 1.0)
    cy = jnp.where(i < NBMID, INV_SQRT2, 0.0)
    y = pred[...] * cself + y1[...] * cy
    out[...] = lax.dot_general(
        y, Wm[...], (((1,), (1,)), ((), ())),
        preferred_element_type=jnp.float32) + bm[...]


_pred_call = pl.pallas_call(
    _pred_body,
    grid=(NBP,),
    in_specs=[
        pl.BlockSpec((BR, D), lambda i: (i, 0)),
        pl.BlockSpec((BR, D), lambda i: (jnp.minimum(i + NBO, NBP - 1), 0)),
        pl.BlockSpec((D, D), lambda i: (0, 0)),
        pl.BlockSpec((1, D), lambda i: (0, 0)),
    ],
    out_specs=pl.BlockSpec((BR, D), lambda i: (i, 0)),
    out_shape=jax.ShapeDtypeStruct((T, D), jnp.float32),
)


@jax.jit
def kernel(obj_vecs, pred_vecs, edges, W, b):
    s3 = edges[:, 0].reshape(NW, NCH, CH)
    o3 = edges[:, 1].reshape(NW, NCH, CH)
    bm = b.reshape(1, D)

    hist = _hist_call(o3)
    scaled, scaled2, dinv, invdeg = _prep_call(hist, obj_vecs)
    y1 = _gather_call(scaled, s3)
    accp = _scatter_call(pred_vecs, scaled2, o3)
    out_pred = _pred_call(pred_vecs, y1, W, bm)
    out_obj = _obj_call(obj_vecs, y1, accp, dinv, invdeg, W, bm)
    return out_obj, out_pred
